# Initial kernel scaffold; baseline (speedup 1.0000x reference)
import jax
import jax.numpy as jnp
from jax.experimental import pallas as pl


def kernel(features, adj, batch1, W_self_1, W_neigh_1, W_self_2, W_neigh_2):
    raise NotImplementedError("write your pallas kernel here")


# speedup vs baseline: 3.3255x; 3.3255x over previous
import functools

import jax
import jax.numpy as jnp
import numpy as np
from jax import lax
from jax.experimental import pallas as pl
from jax.experimental.pallas import tpu as pltpu
from jax.experimental.pallas import tpu_sc as plsc

N_NODES = 10000
FEAT = 128
HID = 128
MAX_DEG = 32
B = 512
S1N = 10
S2N = 25

NC = 2
NS = 16
NW = NC * NS
BPW = B // NW
N1W = BPW * S1N
CHN = 5
NCHUNK = N1W // CHN
LANES = 16
ADJV_ROWS = N_NODES * MAX_DEG // 128

_J1 = np.arange(N1W)
_T1R = np.asarray(_J1 // S1N, dtype=np.int32)
_T1C = np.asarray(_J1 % S1N, dtype=np.int32)
_E2 = np.arange(128)
_T2N = np.asarray(np.minimum(_E2 // S2N, CHN - 1), dtype=np.int32)
_T2K = np.asarray(np.where(_E2 // S2N <= CHN - 1, _E2 % S2N, S2N - 1),
                  dtype=np.int32)


def _sc_body(features, adjv, batch1, t1r, t1c, t2n, t2k,
             h0_out, h1_out, s2_out,
             bids, brow, adj1v, idx1, vrow, adj2v, idx2,
             c1r, c1c, c2n, c2k, h0l, h1l, gbuf, s2l, sem):
    wid = lax.axis_index("s") * NC + lax.axis_index("c")
    base_b = wid * BPW

    pltpu.sync_copy(t1r, c1r)
    pltpu.sync_copy(t1c, c1c)
    pltpu.sync_copy(t2n, c2n)
    pltpu.sync_copy(t2k, c2k)

    pltpu.sync_copy(batch1.at[pl.ds(base_b, BPW)], bids)

    brow[...] = lax.shift_right_arithmetic(bids[...], 2)
    pltpu.async_copy(adjv.at[brow], adj1v, sem).wait()
    pltpu.async_copy(features.at[bids], h0l, sem).wait()
    pltpu.sync_copy(h0l, h0_out.at[pl.ds(base_b, BPW)])

    for t in range(N1W // LANES):
        sl = pl.ds(t * LANES, LANES)
        r = c1r[sl]
        bv = bids[...]
        node = bv.at[r].get(mode="promise_in_bounds")
        col = lax.shift_left(jnp.bitwise_and(node, 3), 5) + c1c[sl]
        idx1[sl] = plsc.load_gather(adj1v, [r, col])
        vrow[sl] = lax.shift_right_arithmetic(idx1[sl], 2)

    for h in range(2):
        sl = pl.ds(h * (N1W // 2), N1W // 2)
        pltpu.async_copy(adjv.at[vrow.at[sl]], adj2v.at[sl], sem).wait()
        pltpu.async_copy(features.at[idx1.at[sl]], h1l.at[sl], sem).wait()
    pltpu.sync_copy(h1l, h1_out.at[pl.ds(wid * N1W, N1W)])

    def chunk_body(ch, _):
        for t in range(8):
            sl = pl.ds(t * LANES, LANES)
            p = ch * CHN + c2n[sl]
            node = plsc.load_gather(idx1, [p])
            col = lax.shift_left(jnp.bitwise_and(node, 3), 5) + c2k[sl]
            idx2[sl] = plsc.load_gather(adj2v, [p, col])
        pltpu.async_copy(features.at[idx2], gbuf, sem).wait()
        for i in range(CHN):
            for d in range(FEAT // LANES):
                acc = gbuf[i * S2N, pl.ds(d * LANES, LANES)]
                for r in range(1, S2N):
                    acc = acc + gbuf[i * S2N + r, pl.ds(d * LANES, LANES)]
                s2l[pl.ds(((ch * CHN + i) * FEAT) + d * LANES, LANES)] = acc
        return ()

    lax.fori_loop(0, NCHUNK, chunk_body, (), unroll=False)
    pltpu.sync_copy(s2l, s2_out.at[pl.ds(wid * N1W * FEAT, N1W * FEAT)])


@jax.jit
def _sc_gather(features, adjv, batch1):
    mesh = plsc.VectorSubcoreMesh(
        core_axis_name="c", subcore_axis_name="s",
        num_cores=NC, num_subcores=NS)
    f = pl.kernel(
        _sc_body,
        out_type=(
            jax.ShapeDtypeStruct((B, FEAT), jnp.float32),
            jax.ShapeDtypeStruct((B * S1N, FEAT), jnp.float32),
            jax.ShapeDtypeStruct((B * S1N * FEAT,), jnp.float32),
        ),
        mesh=mesh,
        scratch_types=[
            pltpu.VMEM((BPW,), jnp.int32),
            pltpu.VMEM((BPW,), jnp.int32),
            pltpu.VMEM((BPW, 128), jnp.int32),
            pltpu.VMEM((N1W,), jnp.int32),
            pltpu.VMEM((N1W,), jnp.int32),
            pltpu.VMEM((N1W, 128), jnp.int32),
            pltpu.VMEM((128,), jnp.int32),
            pltpu.VMEM((N1W,), jnp.int32),
            pltpu.VMEM((N1W,), jnp.int32),
            pltpu.VMEM((128,), jnp.int32),
            pltpu.VMEM((128,), jnp.int32),
            pltpu.VMEM((BPW, FEAT), jnp.float32),
            pltpu.VMEM((N1W, FEAT), jnp.float32),
            pltpu.VMEM((128, FEAT), jnp.float32),
            pltpu.VMEM((N1W * FEAT,), jnp.float32),
            pltpu.SemaphoreType.DMA,
        ],
        compiler_params=pltpu.CompilerParams(needs_layout_passes=False),
    )
    return f(features, adjv, batch1, _T1R, _T1C, _T2N, _T2K)


def _tc_body(h0_ref, h1_ref, s2_ref, ws1_ref, wn1_ref, ws2_ref, wn2_ref,
             out_ref):
    f32 = jnp.float32
    h1 = h1_ref[...]
    s2 = s2_ref[...] * (1.0 / S2N)
    ws1 = ws1_ref[...]
    wn1 = wn1_ref[...]
    a1a = jnp.maximum(jnp.dot(h1, ws1, preferred_element_type=f32), 0.0)
    a1b = jnp.maximum(jnp.dot(s2, wn1, preferred_element_type=f32), 0.0)
    m1a = jnp.mean(a1a.reshape(B, S1N, HID), axis=1)
    m1b = jnp.mean(a1b.reshape(B, S1N, HID), axis=1)
    s1 = jnp.mean(h1.reshape(B, S1N, FEAT), axis=1)
    h0 = h0_ref[...]
    a0a = jnp.maximum(jnp.dot(h0, ws1, preferred_element_type=f32), 0.0)
    a0b = jnp.maximum(jnp.dot(s1, wn1, preferred_element_type=f32), 0.0)
    o1 = (jnp.dot(a0a, ws2_ref[0:HID, :], preferred_element_type=f32)
          + jnp.dot(a0b, ws2_ref[HID:2 * HID, :], preferred_element_type=f32))
    o2 = (jnp.dot(m1a, wn2_ref[0:HID, :], preferred_element_type=f32)
          + jnp.dot(m1b, wn2_ref[HID:2 * HID, :], preferred_element_type=f32))
    nrm = jnp.sqrt(jnp.maximum(
        jnp.sum(o1 * o1 + o2 * o2, axis=1, keepdims=True), 1e-12))
    inv = 1.0 / nrm
    out_ref[:, 0:HID] = o1 * inv
    out_ref[:, HID:2 * HID] = o2 * inv


def _tc_aggregate(h0, h1, s2sum, ws1, wn1, ws2, wn2, interpret=False):
    return pl.pallas_call(
        _tc_body,
        out_shape=jax.ShapeDtypeStruct((B, 2 * HID), jnp.float32),
        interpret=interpret,
    )(h0, h1, s2sum, ws1, wn1, ws2, wn2)


def kernel(features, adj, batch1, W_self_1, W_neigh_1, W_self_2, W_neigh_2):
    adjv = adj.reshape(ADJV_ROWS, 128)
    h0, h1, s2flat = _sc_gather(features, adjv, batch1)
    s2sum = s2flat.reshape(B * S1N, FEAT)
    return _tc_aggregate(h0, h1, s2sum, W_self_1, W_neigh_1, W_self_2,
                         W_neigh_2)


# speedup vs baseline: 3.9882x; 1.1993x over previous
import functools

import jax
import jax.numpy as jnp
import numpy as np
from jax import lax
from jax.experimental import pallas as pl
from jax.experimental.pallas import tpu as pltpu
from jax.experimental.pallas import tpu_sc as plsc

N_NODES = 10000
FEAT = 128
HID = 128
MAX_DEG = 32
B = 512
S1N = 10
S2N = 25

NC = 2
NS = 16
NW = NC * NS
BPW = B // NW
N1W = BPW * S1N
CHN = 5
NCHUNK = N1W // CHN
LANES = 16
ADJV_ROWS = N_NODES * MAX_DEG // 128

_J1 = np.arange(N1W)
_T1R = np.asarray(_J1 // S1N, dtype=np.int32)
_T1C = np.asarray(_J1 % S1N, dtype=np.int32)
_E2 = np.arange(128)
_T2N = np.asarray(np.minimum(_E2 // S2N, CHN - 1), dtype=np.int32)
_T2K = np.asarray(np.where(_E2 // S2N <= CHN - 1, _E2 % S2N, S2N - 1),
                  dtype=np.int32)


def _sc_body(features, adjv, batch1, t1r, t1c, t2n, t2k,
             h0_out, h1_out, s2_out,
             bids, brow, adj1v, idx1, vrow, adj2v, idx2all,
             c1r, c1c, c2n, c2k, h0l, h1l, gbuf, gbuf2, s2l, sem, sem2):
    wid = lax.axis_index("s") * NC + lax.axis_index("c")
    base_b = wid * BPW

    pltpu.sync_copy(t1r, c1r)
    pltpu.sync_copy(t1c, c1c)
    pltpu.sync_copy(t2n, c2n)
    pltpu.sync_copy(t2k, c2k)

    pltpu.sync_copy(batch1.at[pl.ds(base_b, BPW)], bids)

    brow[...] = lax.shift_right_arithmetic(bids[...], 2)
    pltpu.async_copy(adjv.at[brow], adj1v, sem).wait()
    pltpu.async_copy(features.at[bids], h0l, sem).wait()
    pltpu.sync_copy(h0l, h0_out.at[pl.ds(base_b, BPW)])

    for t in range(N1W // LANES):
        sl = pl.ds(t * LANES, LANES)
        r = c1r[sl]
        bv = bids[...]
        node = bv.at[r].get(mode="promise_in_bounds")
        col = lax.shift_left(jnp.bitwise_and(node, 3), 5) + c1c[sl]
        idx1[sl] = plsc.load_gather(adj1v, [r, col])
        vrow[sl] = lax.shift_right_arithmetic(idx1[sl], 2)

    for h in range(2):
        sl = pl.ds(h * (N1W // 2), N1W // 2)
        pltpu.async_copy(adjv.at[vrow.at[sl]], adj2v.at[sl], sem).wait()
        pltpu.async_copy(features.at[idx1.at[sl]], h1l.at[sl], sem).wait()
    pltpu.sync_copy(h1l, h1_out.at[pl.ds(wid * N1W, N1W)])

    def ext_body(tg, _):
        ch = lax.shift_right_logical(tg, 3)
        off = jnp.bitwise_and(tg, 7) * LANES
        sl = pl.ds(off, LANES)
        p = ch * CHN + c2n[sl]
        node = plsc.load_gather(idx1, [p])
        col = lax.shift_left(jnp.bitwise_and(node, 3), 5) + c2k[sl]
        idx2all[pl.ds(tg * LANES, LANES)] = plsc.load_gather(adj2v, [p, col])
        return ()

    lax.fori_loop(0, NCHUNK * 8, ext_body, (), unroll=False)

    def issue(ch, buf, s):
        return pltpu.async_copy(
            features.at[idx2all.at[pl.ds(ch * 128, 128)]], buf, s)

    def reduce(buf, ch):
        for i in range(CHN):
            for d in range(FEAT // LANES):
                acc = buf[i * S2N, pl.ds(d * LANES, LANES)]
                for r in range(1, S2N):
                    acc = acc + buf[i * S2N + r, pl.ds(d * LANES, LANES)]
                s2l[pl.ds(((ch * CHN + i) * FEAT) + d * LANES, LANES)] = acc

    issue(0, gbuf, sem)

    def wait_for(ch, buf, s):
        pltpu.make_async_copy(
            features.at[idx2all.at[pl.ds(ch * 128, 128)]], buf, s).wait()

    def pair_body(g, _):
        ch = g * 2
        issue(ch + 1, gbuf2, sem2)
        wait_for(ch, gbuf, sem)
        reduce(gbuf, ch)

        @pl.when(ch + 2 < NCHUNK)
        def _():
            issue(ch + 2, gbuf, sem)

        wait_for(ch + 1, gbuf2, sem2)
        reduce(gbuf2, ch + 1)
        return ()

    lax.fori_loop(0, NCHUNK // 2, pair_body, (), unroll=False)
    pltpu.sync_copy(s2l, s2_out.at[pl.ds(wid * N1W * FEAT, N1W * FEAT)])


@jax.jit
def _sc_gather(features, adjv, batch1):
    mesh = plsc.VectorSubcoreMesh(
        core_axis_name="c", subcore_axis_name="s",
        num_cores=NC, num_subcores=NS)
    f = pl.kernel(
        _sc_body,
        out_type=(
            jax.ShapeDtypeStruct((B, FEAT), jnp.float32),
            jax.ShapeDtypeStruct((B * S1N, FEAT), jnp.float32),
            jax.ShapeDtypeStruct((B * S1N * FEAT,), jnp.float32),
        ),
        mesh=mesh,
        scratch_types=[
            pltpu.VMEM((BPW,), jnp.int32),
            pltpu.VMEM((BPW,), jnp.int32),
            pltpu.VMEM((BPW, 128), jnp.int32),
            pltpu.VMEM((N1W,), jnp.int32),
            pltpu.VMEM((N1W,), jnp.int32),
            pltpu.VMEM((N1W, 128), jnp.int32),
            pltpu.VMEM((NCHUNK * 128,), jnp.int32),
            pltpu.VMEM((N1W,), jnp.int32),
            pltpu.VMEM((N1W,), jnp.int32),
            pltpu.VMEM((128,), jnp.int32),
            pltpu.VMEM((128,), jnp.int32),
            pltpu.VMEM((BPW, FEAT), jnp.float32),
            pltpu.VMEM((N1W, FEAT), jnp.float32),
            pltpu.VMEM((128, FEAT), jnp.float32),
            pltpu.VMEM((128, FEAT), jnp.float32),
            pltpu.VMEM((N1W * FEAT,), jnp.float32),
            pltpu.SemaphoreType.DMA,
            pltpu.SemaphoreType.DMA,
        ],
        compiler_params=pltpu.CompilerParams(needs_layout_passes=False),
    )
    return f(features, adjv, batch1, _T1R, _T1C, _T2N, _T2K)


def _tc_body(h0_ref, h1_ref, s2_ref, ws1_ref, wn1_ref, ws2_ref, wn2_ref,
             out_ref):
    f32 = jnp.float32
    h1 = h1_ref[...]
    s2 = s2_ref[...] * (1.0 / S2N)
    ws1 = ws1_ref[...]
    wn1 = wn1_ref[...]
    a1a = jnp.maximum(jnp.dot(h1, ws1, preferred_element_type=f32), 0.0)
    a1b = jnp.maximum(jnp.dot(s2, wn1, preferred_element_type=f32), 0.0)
    m1a = jnp.mean(a1a.reshape(B, S1N, HID), axis=1)
    m1b = jnp.mean(a1b.reshape(B, S1N, HID), axis=1)
    s1 = jnp.mean(h1.reshape(B, S1N, FEAT), axis=1)
    h0 = h0_ref[...]
    a0a = jnp.maximum(jnp.dot(h0, ws1, preferred_element_type=f32), 0.0)
    a0b = jnp.maximum(jnp.dot(s1, wn1, preferred_element_type=f32), 0.0)
    o1 = (jnp.dot(a0a, ws2_ref[0:HID, :], preferred_element_type=f32)
          + jnp.dot(a0b, ws2_ref[HID:2 * HID, :], preferred_element_type=f32))
    o2 = (jnp.dot(m1a, wn2_ref[0:HID, :], preferred_element_type=f32)
          + jnp.dot(m1b, wn2_ref[HID:2 * HID, :], preferred_element_type=f32))
    nrm = jnp.sqrt(jnp.maximum(
        jnp.sum(o1 * o1 + o2 * o2, axis=1, keepdims=True), 1e-12))
    inv = 1.0 / nrm
    out_ref[:, 0:HID] = o1 * inv
    out_ref[:, HID:2 * HID] = o2 * inv


def _tc_aggregate(h0, h1, s2sum, ws1, wn1, ws2, wn2, interpret=False):
    return pl.pallas_call(
        _tc_body,
        out_shape=jax.ShapeDtypeStruct((B, 2 * HID), jnp.float32),
        interpret=interpret,
    )(h0, h1, s2sum, ws1, wn1, ws2, wn2)


def kernel(features, adj, batch1, W_self_1, W_neigh_1, W_self_2, W_neigh_2):
    adjv = adj.reshape(ADJV_ROWS, 128)
    h0, h1, s2flat = _sc_gather(features, adjv, batch1)
    s2sum = s2flat.reshape(B * S1N, FEAT)
    return _tc_aggregate(h0, h1, s2sum, W_self_1, W_neigh_1, W_self_2,
                         W_neigh_2)


# speedup vs baseline: 5.0300x; 1.2612x over previous
import functools

import jax
import jax.numpy as jnp
import numpy as np
from jax import lax
from jax.experimental import pallas as pl
from jax.experimental.pallas import tpu as pltpu
from jax.experimental.pallas import tpu_sc as plsc

N_NODES = 10000
FEAT = 128
HID = 128
MAX_DEG = 32
B = 512
S1N = 10
S2N = 25

NC = 2
NS = 16
NW = NC * NS
BPW = B // NW
N1W = BPW * S1N
CHN = 5
NCHUNK = N1W // CHN
LANES = 16
ADJV_ROWS = N_NODES * MAX_DEG // 128

_J1 = np.arange(N1W)
_T1R = np.asarray(_J1 // S1N, dtype=np.int32)
_T1C = np.asarray(_J1 % S1N, dtype=np.int32)
_E2 = np.arange(128)
_T2N = np.asarray(np.minimum(_E2 // S2N, CHN - 1), dtype=np.int32)
_T2K = np.asarray(np.where(_E2 // S2N <= CHN - 1, _E2 % S2N, S2N - 1),
                  dtype=np.int32)


def _sc_body(features, adjv, batch1, t1r, t1c, t2n, t2k,
             h0_out, h1_out, s2_out,
             bids, brow, adj1v, idx1, vrow, adj2v, idx2all,
             c1r, c1c, c2n, c2k, h0l, h1l, gbuf, gbuf2, s2l, sem, sem2):
    wid = lax.axis_index("s") * NC + lax.axis_index("c")
    base_b = wid * BPW

    pltpu.sync_copy(t1r, c1r)
    pltpu.sync_copy(t1c, c1c)
    pltpu.sync_copy(t2n, c2n)
    pltpu.sync_copy(t2k, c2k)

    pltpu.sync_copy(batch1.at[pl.ds(base_b, BPW)], bids)

    brow[...] = lax.shift_right_arithmetic(bids[...], 2)
    pltpu.async_copy(adjv.at[brow], adj1v, sem).wait()
    pltpu.async_copy(features.at[bids], h0l, sem).wait()
    pltpu.sync_copy(h0l, h0_out.at[pl.ds(base_b, BPW)])

    for t in range(N1W // LANES):
        sl = pl.ds(t * LANES, LANES)
        r = c1r[sl]
        bv = bids[...]
        node = bv.at[r].get(mode="promise_in_bounds")
        col = lax.shift_left(jnp.bitwise_and(node, 3), 5) + c1c[sl]
        idx1[sl] = plsc.load_gather(adj1v, [r, col])
        vrow[sl] = lax.shift_right_arithmetic(idx1[sl], 2)

    for h in range(2):
        sl = pl.ds(h * (N1W // 2), N1W // 2)
        pltpu.async_copy(adjv.at[vrow.at[sl]], adj2v.at[sl], sem).wait()
        pltpu.async_copy(features.at[idx1.at[sl]], h1l.at[sl], sem).wait()
    pltpu.sync_copy(h1l, h1_out.at[pl.ds(wid * N1W, N1W)])

    def ext_body(tg, _):
        ch = lax.shift_right_logical(tg, 3)
        off = jnp.bitwise_and(tg, 7) * LANES
        sl = pl.ds(off, LANES)
        p = ch * CHN + c2n[sl]
        node = plsc.load_gather(idx1, [p])
        col = lax.shift_left(jnp.bitwise_and(node, 3), 5) + c2k[sl]
        idx2all[pl.ds(tg * LANES, LANES)] = plsc.load_gather(adj2v, [p, col])
        return ()

    lax.fori_loop(0, NCHUNK * 8, ext_body, (), unroll=False)

    def issue(ch, buf, s):
        return pltpu.async_copy(
            features.at[idx2all.at[pl.ds(ch * 128, 128)]], buf, s)

    def reduce(buf, ch):
        for i in range(CHN):
            for d in range(FEAT // LANES):
                sl = pl.ds(d * LANES, LANES)
                v = [buf[i * S2N + r, sl] for r in range(S2N)]
                while len(v) > 1:
                    nxt = [v[k] + v[k + 1] for k in range(0, len(v) - 1, 2)]
                    if len(v) % 2:
                        nxt.append(v[-1])
                    v = nxt
                s2l[pl.ds(((ch * CHN + i) * FEAT) + d * LANES, LANES)] = v[0]

    issue(0, gbuf, sem)

    def wait_for(ch, buf, s):
        pltpu.make_async_copy(
            features.at[idx2all.at[pl.ds(ch * 128, 128)]], buf, s).wait()

    def pair_body(g, _):
        ch = g * 2
        issue(ch + 1, gbuf2, sem2)
        wait_for(ch, gbuf, sem)
        reduce(gbuf, ch)

        @pl.when(ch + 2 < NCHUNK)
        def _():
            issue(ch + 2, gbuf, sem)

        wait_for(ch + 1, gbuf2, sem2)
        reduce(gbuf2, ch + 1)
        return ()

    lax.fori_loop(0, NCHUNK // 2, pair_body, (), unroll=False)
    pltpu.sync_copy(s2l, s2_out.at[pl.ds(wid * N1W * FEAT, N1W * FEAT)])


@jax.jit
def _sc_gather(features, adjv, batch1):
    mesh = plsc.VectorSubcoreMesh(
        core_axis_name="c", subcore_axis_name="s",
        num_cores=NC, num_subcores=NS)
    f = pl.kernel(
        _sc_body,
        out_type=(
            jax.ShapeDtypeStruct((B, FEAT), jnp.float32),
            jax.ShapeDtypeStruct((B * S1N, FEAT), jnp.float32),
            jax.ShapeDtypeStruct((B * S1N * FEAT,), jnp.float32),
        ),
        mesh=mesh,
        scratch_types=[
            pltpu.VMEM((BPW,), jnp.int32),
            pltpu.VMEM((BPW,), jnp.int32),
            pltpu.VMEM((BPW, 128), jnp.int32),
            pltpu.VMEM((N1W,), jnp.int32),
            pltpu.VMEM((N1W,), jnp.int32),
            pltpu.VMEM((N1W, 128), jnp.int32),
            pltpu.VMEM((NCHUNK * 128,), jnp.int32),
            pltpu.VMEM((N1W,), jnp.int32),
            pltpu.VMEM((N1W,), jnp.int32),
            pltpu.VMEM((128,), jnp.int32),
            pltpu.VMEM((128,), jnp.int32),
            pltpu.VMEM((BPW, FEAT), jnp.float32),
            pltpu.VMEM((N1W, FEAT), jnp.float32),
            pltpu.VMEM((128, FEAT), jnp.float32),
            pltpu.VMEM((128, FEAT), jnp.float32),
            pltpu.VMEM((N1W * FEAT,), jnp.float32),
            pltpu.SemaphoreType.DMA,
            pltpu.SemaphoreType.DMA,
        ],
        compiler_params=pltpu.CompilerParams(needs_layout_passes=False),
    )
    return f(features, adjv, batch1, _T1R, _T1C, _T2N, _T2K)


def _tc_body(h0_ref, h1_ref, s2_ref, ws1_ref, wn1_ref, ws2_ref, wn2_ref,
             out_ref):
    f32 = jnp.float32
    h1 = h1_ref[...]
    s2 = s2_ref[...] * (1.0 / S2N)
    ws1 = ws1_ref[...]
    wn1 = wn1_ref[...]
    a1a = jnp.maximum(jnp.dot(h1, ws1, preferred_element_type=f32), 0.0)
    a1b = jnp.maximum(jnp.dot(s2, wn1, preferred_element_type=f32), 0.0)
    m1a = jnp.mean(a1a.reshape(B, S1N, HID), axis=1)
    m1b = jnp.mean(a1b.reshape(B, S1N, HID), axis=1)
    s1 = jnp.mean(h1.reshape(B, S1N, FEAT), axis=1)
    h0 = h0_ref[...]
    a0a = jnp.maximum(jnp.dot(h0, ws1, preferred_element_type=f32), 0.0)
    a0b = jnp.maximum(jnp.dot(s1, wn1, preferred_element_type=f32), 0.0)
    o1 = (jnp.dot(a0a, ws2_ref[0:HID, :], preferred_element_type=f32)
          + jnp.dot(a0b, ws2_ref[HID:2 * HID, :], preferred_element_type=f32))
    o2 = (jnp.dot(m1a, wn2_ref[0:HID, :], preferred_element_type=f32)
          + jnp.dot(m1b, wn2_ref[HID:2 * HID, :], preferred_element_type=f32))
    nrm = jnp.sqrt(jnp.maximum(
        jnp.sum(o1 * o1 + o2 * o2, axis=1, keepdims=True), 1e-12))
    inv = 1.0 / nrm
    out_ref[:, 0:HID] = o1 * inv
    out_ref[:, HID:2 * HID] = o2 * inv


def _tc_aggregate(h0, h1, s2sum, ws1, wn1, ws2, wn2, interpret=False):
    return pl.pallas_call(
        _tc_body,
        out_shape=jax.ShapeDtypeStruct((B, 2 * HID), jnp.float32),
        interpret=interpret,
    )(h0, h1, s2sum, ws1, wn1, ws2, wn2)


def kernel(features, adj, batch1, W_self_1, W_neigh_1, W_self_2, W_neigh_2):
    adjv = adj.reshape(ADJV_ROWS, 128)
    h0, h1, s2flat = _sc_gather(features, adjv, batch1)
    s2sum = s2flat.reshape(B * S1N, FEAT)
    return _tc_aggregate(h0, h1, s2sum, W_self_1, W_neigh_1, W_self_2,
                         W_neigh_2)


# speedup vs baseline: 5.1520x; 1.0243x over previous
import functools

import jax
import jax.numpy as jnp
import numpy as np
from jax import lax
from jax.experimental import pallas as pl
from jax.experimental.pallas import tpu as pltpu
from jax.experimental.pallas import tpu_sc as plsc

N_NODES = 10000
FEAT = 128
HID = 128
MAX_DEG = 32
B = 512
S1N = 10
S2N = 25

NC = 2
NS = 16
NW = NC * NS
BPW = B // NW
N1W = BPW * S1N
CHN = 5
NCHUNK = N1W // CHN
LANES = 16
ADJV_ROWS = N_NODES * MAX_DEG // 128

_J1 = np.arange(N1W)
_T1R = np.asarray(_J1 // S1N, dtype=np.int32)
_T1C = np.asarray(_J1 % S1N, dtype=np.int32)
_E2 = np.arange(128)
_T2N = np.asarray(np.minimum(_E2 // S2N, CHN - 1), dtype=np.int32)
_T2K = np.asarray(np.where(_E2 // S2N <= CHN - 1, _E2 % S2N, S2N - 1),
                  dtype=np.int32)


def _sc_body(features, adjv, batch1, t1r, t1c, t2n, t2k,
             h0_out, h1_out, s2_out,
             bids, brow, adj1v, idx1, vrow, adj2v, idx2all,
             c1r, c1c, c2n, c2k, h0l, h1l, gbuf, gbuf2, s2l, sem, sem2, sem3, sem4):
    wid = lax.axis_index("s") * NC + lax.axis_index("c")
    base_b = wid * BPW

    pltpu.sync_copy(t1r, c1r)
    pltpu.sync_copy(t1c, c1c)
    pltpu.sync_copy(t2n, c2n)
    pltpu.sync_copy(t2k, c2k)

    pltpu.sync_copy(batch1.at[pl.ds(base_b, BPW)], bids)

    brow[...] = lax.shift_right_arithmetic(bids[...], 2)
    adj1_cp = pltpu.async_copy(adjv.at[brow], adj1v, sem)
    h0_cp = pltpu.async_copy(features.at[bids], h0l, sem4)
    adj1_cp.wait()

    for t in range(N1W // LANES):
        sl = pl.ds(t * LANES, LANES)
        r = c1r[sl]
        bv = bids[...]
        node = bv.at[r].get(mode="promise_in_bounds")
        col = lax.shift_left(jnp.bitwise_and(node, 3), 5) + c1c[sl]
        idx1[sl] = plsc.load_gather(adj1v, [r, col])
        vrow[sl] = lax.shift_right_arithmetic(idx1[sl], 2)

    adj2_cps = []
    h1_cps = []
    for h, s_adj, s_h1 in ((0, sem, sem2), (1, sem3, sem2)):
        sl = pl.ds(h * (N1W // 2), N1W // 2)
        adj2_cps.append(pltpu.async_copy(adjv.at[vrow.at[sl]],
                                         adj2v.at[sl], s_adj))
        h1_cps.append(pltpu.async_copy(features.at[idx1.at[sl]],
                                       h1l.at[sl], s_h1))
    h0_cp.wait()
    pltpu.sync_copy(h0l, h0_out.at[pl.ds(base_b, BPW)])
    for cp in adj2_cps:
        cp.wait()

    def ext_body(tg, _):
        ch = lax.shift_right_logical(tg, 3)
        off = jnp.bitwise_and(tg, 7) * LANES
        sl = pl.ds(off, LANES)
        p = ch * CHN + c2n[sl]
        node = plsc.load_gather(idx1, [p])
        col = lax.shift_left(jnp.bitwise_and(node, 3), 5) + c2k[sl]
        idx2all[pl.ds(tg * LANES, LANES)] = plsc.load_gather(adj2v, [p, col])
        return ()

    lax.fori_loop(0, NCHUNK * 8, ext_body, (), unroll=False)
    for cp in h1_cps:
        cp.wait()
    pltpu.sync_copy(h1l, h1_out.at[pl.ds(wid * N1W, N1W)])

    def issue(ch, buf, s):
        return pltpu.async_copy(
            features.at[idx2all.at[pl.ds(ch * 128, 128)]], buf, s)

    def reduce(buf, ch):
        for i in range(CHN):
            for d in range(FEAT // LANES):
                sl = pl.ds(d * LANES, LANES)
                v = [buf[i * S2N + r, sl] for r in range(S2N)]
                while len(v) > 1:
                    nxt = [v[k] + v[k + 1] for k in range(0, len(v) - 1, 2)]
                    if len(v) % 2:
                        nxt.append(v[-1])
                    v = nxt
                s2l[ch * CHN + i, pl.ds(d * LANES, LANES)] = v[0]

    issue(0, gbuf, sem)

    def wait_for(ch, buf, s):
        pltpu.make_async_copy(
            features.at[idx2all.at[pl.ds(ch * 128, 128)]], buf, s).wait()

    def pair_body(g, _):
        ch = g * 2
        issue(ch + 1, gbuf2, sem2)
        wait_for(ch, gbuf, sem)
        reduce(gbuf, ch)

        @pl.when(ch + 2 < NCHUNK)
        def _():
            issue(ch + 2, gbuf, sem)

        wait_for(ch + 1, gbuf2, sem2)
        reduce(gbuf2, ch + 1)
        return ()

    lax.fori_loop(0, NCHUNK // 2, pair_body, (), unroll=False)
    pltpu.sync_copy(s2l, s2_out.at[pl.ds(wid * N1W, N1W)])


@jax.jit
def _sc_gather(features, adjv, batch1):
    mesh = plsc.VectorSubcoreMesh(
        core_axis_name="c", subcore_axis_name="s",
        num_cores=NC, num_subcores=NS)
    f = pl.kernel(
        _sc_body,
        out_type=(
            jax.ShapeDtypeStruct((B, FEAT), jnp.float32),
            jax.ShapeDtypeStruct((B * S1N, FEAT), jnp.float32),
            jax.ShapeDtypeStruct((B * S1N, FEAT), jnp.float32),
        ),
        mesh=mesh,
        scratch_types=[
            pltpu.VMEM((BPW,), jnp.int32),
            pltpu.VMEM((BPW,), jnp.int32),
            pltpu.VMEM((BPW, 128), jnp.int32),
            pltpu.VMEM((N1W,), jnp.int32),
            pltpu.VMEM((N1W,), jnp.int32),
            pltpu.VMEM((N1W, 128), jnp.int32),
            pltpu.VMEM((NCHUNK * 128,), jnp.int32),
            pltpu.VMEM((N1W,), jnp.int32),
            pltpu.VMEM((N1W,), jnp.int32),
            pltpu.VMEM((128,), jnp.int32),
            pltpu.VMEM((128,), jnp.int32),
            pltpu.VMEM((BPW, FEAT), jnp.float32),
            pltpu.VMEM((N1W, FEAT), jnp.float32),
            pltpu.VMEM((128, FEAT), jnp.float32),
            pltpu.VMEM((128, FEAT), jnp.float32),
            pltpu.VMEM((N1W, FEAT), jnp.float32),
            pltpu.SemaphoreType.DMA,
            pltpu.SemaphoreType.DMA,
            pltpu.SemaphoreType.DMA,
            pltpu.SemaphoreType.DMA,
        ],
        compiler_params=pltpu.CompilerParams(needs_layout_passes=False),
    )
    return f(features, adjv, batch1, _T1R, _T1C, _T2N, _T2K)


def _tc_body(h0_ref, h1_ref, s2_ref, ws1_ref, wn1_ref, ws2_ref, wn2_ref,
             out_ref):
    f32 = jnp.float32
    h1 = h1_ref[...]
    s2 = s2_ref[...] * (1.0 / S2N)
    ws1 = ws1_ref[...]
    wn1 = wn1_ref[...]
    a1a = jnp.maximum(jnp.dot(h1, ws1, preferred_element_type=f32), 0.0)
    a1b = jnp.maximum(jnp.dot(s2, wn1, preferred_element_type=f32), 0.0)
    m1a = jnp.mean(a1a.reshape(B, S1N, HID), axis=1)
    m1b = jnp.mean(a1b.reshape(B, S1N, HID), axis=1)
    s1 = jnp.mean(h1.reshape(B, S1N, FEAT), axis=1)
    h0 = h0_ref[...]
    a0a = jnp.maximum(jnp.dot(h0, ws1, preferred_element_type=f32), 0.0)
    a0b = jnp.maximum(jnp.dot(s1, wn1, preferred_element_type=f32), 0.0)
    o1 = (jnp.dot(a0a, ws2_ref[0:HID, :], preferred_element_type=f32)
          + jnp.dot(a0b, ws2_ref[HID:2 * HID, :], preferred_element_type=f32))
    o2 = (jnp.dot(m1a, wn2_ref[0:HID, :], preferred_element_type=f32)
          + jnp.dot(m1b, wn2_ref[HID:2 * HID, :], preferred_element_type=f32))
    nrm = jnp.sqrt(jnp.maximum(
        jnp.sum(o1 * o1 + o2 * o2, axis=1, keepdims=True), 1e-12))
    inv = 1.0 / nrm
    out_ref[:, 0:HID] = o1 * inv
    out_ref[:, HID:2 * HID] = o2 * inv


def _tc_aggregate(h0, h1, s2sum, ws1, wn1, ws2, wn2, interpret=False):
    return pl.pallas_call(
        _tc_body,
        out_shape=jax.ShapeDtypeStruct((B, 2 * HID), jnp.float32),
        interpret=interpret,
    )(h0, h1, s2sum, ws1, wn1, ws2, wn2)


def kernel(features, adj, batch1, W_self_1, W_neigh_1, W_self_2, W_neigh_2):
    adjv = adj.reshape(ADJV_ROWS, 128)
    h0, h1, s2sum = _sc_gather(features, adjv, batch1)
    return _tc_aggregate(h0, h1, s2sum, W_self_1, W_neigh_1, W_self_2,
                         W_neigh_2)


# speedup vs baseline: 7.1735x; 1.3924x over previous
import functools

import jax
import jax.numpy as jnp
import numpy as np
from jax import lax
from jax.experimental import pallas as pl
from jax.experimental.pallas import tpu as pltpu
from jax.experimental.pallas import tpu_sc as plsc

N_NODES = 10000
FEAT = 128
HID = 128
MAX_DEG = 32
B = 512
S1N = 10
S2N = 25

NC = 2
NS = 16
NW = NC * NS
BPW = B // NW
N1W = BPW * S1N
CHN = 5
NCHUNK = N1W // CHN
LANES = 16
ADJV_ROWS = N_NODES * MAX_DEG // 128

_J1 = np.arange(N1W)
_T1R = np.asarray(_J1 // S1N, dtype=np.int32)
_T1C = np.asarray(_J1 % S1N, dtype=np.int32)
_E2 = np.arange(128)
_T2N = np.asarray(np.minimum(_E2 // S2N, CHN - 1), dtype=np.int32)
_T2K = np.asarray(np.where(_E2 // S2N <= CHN - 1, _E2 % S2N, S2N - 1),
                  dtype=np.int32)
_T2P = np.asarray(np.where(_E2 // S2N <= CHN - 1, _E2 // S2N, 100000),
                  dtype=np.int32)


def _sc_body(features, adjv, batch1, t1r, t1c, t2n, t2k, t2p,
             h0_out, h1_out, s2_out,
             bids, brow, adj1v, idx1, vrow, adj2v, idx2all,
             c1r, c1c, c2n, c2k, c2p, dmapA, dmapB, h0l, h1l, gbuf, gbuf2,
             shacc, sem, sem2, sem3, sem4):
    wid = lax.axis_index("s") * NC + lax.axis_index("c")
    base_b = wid * BPW

    pltpu.sync_copy(t1r, c1r)
    pltpu.sync_copy(t1c, c1c)
    pltpu.sync_copy(t2n, c2n)
    pltpu.sync_copy(t2k, c2k)
    pltpu.sync_copy(t2p, c2p)

    pltpu.sync_copy(batch1.at[pl.ds(base_b, BPW)], bids)

    brow[...] = lax.shift_right_arithmetic(bids[...], 2)
    adj1_cp = pltpu.async_copy(adjv.at[brow], adj1v, sem)
    h0_cp = pltpu.async_copy(features.at[bids], h0l, sem4)
    adj1_cp.wait()

    for t in range(N1W // LANES):
        sl = pl.ds(t * LANES, LANES)
        r = c1r[sl]
        bv = bids[...]
        node = bv.at[r].get(mode="promise_in_bounds")
        col = lax.shift_left(jnp.bitwise_and(node, 3), 5) + c1c[sl]
        idx1[sl] = plsc.load_gather(adj1v, [r, col])
        vrow[sl] = lax.shift_right_arithmetic(idx1[sl], 2)

    adj2_cps = []
    h1_cps = []
    for h, s_adj, s_h1 in ((0, sem, sem2), (1, sem3, sem2)):
        sl = pl.ds(h * (N1W // 2), N1W // 2)
        adj2_cps.append(pltpu.async_copy(adjv.at[vrow.at[sl]],
                                         adj2v.at[sl], s_adj))
        h1_cps.append(pltpu.async_copy(features.at[idx1.at[sl]],
                                       h1l.at[sl], s_h1))
    h0_cp.wait()
    pltpu.sync_copy(h0l, h0_out.at[pl.ds(base_b, BPW)])
    for cp in adj2_cps:
        cp.wait()

    def ext_body(tg, _):
        ch = lax.shift_right_logical(tg, 3)
        off = jnp.bitwise_and(tg, 7) * LANES
        sl = pl.ds(off, LANES)
        p = ch * CHN + c2n[sl]
        node = plsc.load_gather(idx1, [p])
        col = lax.shift_left(jnp.bitwise_and(node, 3), 5) + c2k[sl]
        idx2all[pl.ds(tg * LANES, LANES)] = plsc.load_gather(adj2v, [p, col])
        return ()

    lax.fori_loop(0, NCHUNK * 8, ext_body, (), unroll=False)
    for cp in h1_cps:
        cp.wait()
    pltpu.sync_copy(h1l, h1_out.at[pl.ds(wid * N1W, N1W)])

    sid = lax.axis_index("s")
    slab = sid * (N1W + 1)

    zero = jnp.zeros((LANES,), jnp.float32)
    for rr in range(128):
        for d in range(FEAT // LANES):
            gbuf[rr, pl.ds(d * LANES, LANES)] = zero
    pltpu.sync_copy(gbuf, shacc.at[pl.ds(slab, 128)])
    pltpu.sync_copy(gbuf.at[pl.ds(0, N1W + 1 - 128)],
                    shacc.at[pl.ds(slab + 128, N1W + 1 - 128)])

    def issue(ch, buf, s):
        return pltpu.async_copy(
            features.at[idx2all.at[pl.ds(ch * 128, 128)]], buf, s)

    def wait_for(ch, buf, s):
        pltpu.make_async_copy(
            features.at[idx2all.at[pl.ds(ch * 128, 128)]], buf, s).wait()

    def fill_dmap(ch, dmap):
        for t in range(8):
            sl = pl.ds(t * LANES, LANES)
            dmap[sl] = slab + jnp.minimum(ch * CHN + c2p[sl], N1W)

    issue(0, gbuf, sem)

    def pair_body(g, _):
        ch = g * 2
        issue(ch + 1, gbuf2, sem2)
        fill_dmap(ch, dmapA)
        wait_for(ch, gbuf, sem)
        pltpu.async_copy(gbuf, shacc.at[dmapA], sem3, add=True).wait()

        @pl.when(ch + 2 < NCHUNK)
        def _():
            issue(ch + 2, gbuf, sem)

        fill_dmap(ch + 1, dmapB)
        wait_for(ch + 1, gbuf2, sem2)
        pltpu.async_copy(gbuf2, shacc.at[dmapB], sem4, add=True).wait()
        return ()

    lax.fori_loop(0, NCHUNK // 2, pair_body, (), unroll=False)
    pltpu.sync_copy(shacc.at[pl.ds(slab, N1W)],
                    s2_out.at[pl.ds(wid * N1W, N1W)])


@jax.jit
def _sc_gather(features, adjv, batch1):
    mesh = plsc.VectorSubcoreMesh(
        core_axis_name="c", subcore_axis_name="s",
        num_cores=NC, num_subcores=NS)
    f = pl.kernel(
        _sc_body,
        out_type=(
            jax.ShapeDtypeStruct((B, FEAT), jnp.float32),
            jax.ShapeDtypeStruct((B * S1N, FEAT), jnp.float32),
            jax.ShapeDtypeStruct((B * S1N, FEAT), jnp.float32),
        ),
        mesh=mesh,
        scratch_types=[
            pltpu.VMEM((BPW,), jnp.int32),
            pltpu.VMEM((BPW,), jnp.int32),
            pltpu.VMEM((BPW, 128), jnp.int32),
            pltpu.VMEM((N1W,), jnp.int32),
            pltpu.VMEM((N1W,), jnp.int32),
            pltpu.VMEM((N1W, 128), jnp.int32),
            pltpu.VMEM((NCHUNK * 128,), jnp.int32),
            pltpu.VMEM((N1W,), jnp.int32),
            pltpu.VMEM((N1W,), jnp.int32),
            pltpu.VMEM((128,), jnp.int32),
            pltpu.VMEM((128,), jnp.int32),
            pltpu.VMEM((128,), jnp.int32),
            pltpu.VMEM((128,), jnp.int32),
            pltpu.VMEM((128,), jnp.int32),
            pltpu.VMEM((BPW, FEAT), jnp.float32),
            pltpu.VMEM((N1W, FEAT), jnp.float32),
            pltpu.VMEM((128, FEAT), jnp.float32),
            pltpu.VMEM((128, FEAT), jnp.float32),
            pltpu.VMEM_SHARED((NS * (N1W + 1), FEAT), jnp.float32),
            pltpu.SemaphoreType.DMA,
            pltpu.SemaphoreType.DMA,
            pltpu.SemaphoreType.DMA,
            pltpu.SemaphoreType.DMA,
        ],
        compiler_params=pltpu.CompilerParams(needs_layout_passes=False),
    )
    return f(features, adjv, batch1, _T1R, _T1C, _T2N, _T2K, _T2P)


def _tc_body(h0_ref, h1_ref, s2_ref, ws1_ref, wn1_ref, ws2_ref, wn2_ref,
             out_ref):
    f32 = jnp.float32
    h1 = h1_ref[...]
    s2 = s2_ref[...] * (1.0 / S2N)
    ws1 = ws1_ref[...]
    wn1 = wn1_ref[...]
    a1a = jnp.maximum(jnp.dot(h1, ws1, preferred_element_type=f32), 0.0)
    a1b = jnp.maximum(jnp.dot(s2, wn1, preferred_element_type=f32), 0.0)
    m1a = jnp.mean(a1a.reshape(B, S1N, HID), axis=1)
    m1b = jnp.mean(a1b.reshape(B, S1N, HID), axis=1)
    s1 = jnp.mean(h1.reshape(B, S1N, FEAT), axis=1)
    h0 = h0_ref[...]
    a0a = jnp.maximum(jnp.dot(h0, ws1, preferred_element_type=f32), 0.0)
    a0b = jnp.maximum(jnp.dot(s1, wn1, preferred_element_type=f32), 0.0)
    o1 = (jnp.dot(a0a, ws2_ref[0:HID, :], preferred_element_type=f32)
          + jnp.dot(a0b, ws2_ref[HID:2 * HID, :], preferred_element_type=f32))
    o2 = (jnp.dot(m1a, wn2_ref[0:HID, :], preferred_element_type=f32)
          + jnp.dot(m1b, wn2_ref[HID:2 * HID, :], preferred_element_type=f32))
    nrm = jnp.sqrt(jnp.maximum(
        jnp.sum(o1 * o1 + o2 * o2, axis=1, keepdims=True), 1e-12))
    inv = 1.0 / nrm
    out_ref[:, 0:HID] = o1 * inv
    out_ref[:, HID:2 * HID] = o2 * inv


def _tc_aggregate(h0, h1, s2sum, ws1, wn1, ws2, wn2, interpret=False):
    return pl.pallas_call(
        _tc_body,
        out_shape=jax.ShapeDtypeStruct((B, 2 * HID), jnp.float32),
        interpret=interpret,
    )(h0, h1, s2sum, ws1, wn1, ws2, wn2)


def kernel(features, adj, batch1, W_self_1, W_neigh_1, W_self_2, W_neigh_2):
    adjv = adj.reshape(ADJV_ROWS, 128)
    h0, h1, s2sum = _sc_gather(features, adjv, batch1)
    return _tc_aggregate(h0, h1, s2sum, W_self_1, W_neigh_1, W_self_2,
                         W_neigh_2)
